# Initial kernel scaffold; baseline (speedup 1.0000x reference)
import jax
import jax.numpy as jnp
from jax.experimental import pallas as pl


def kernel(predicate_idx, arg1_idx, arg2_idx, entity_table, predicate_table, W1, b1, W2, b2):
    raise NotImplementedError("write your pallas kernel here")


# speedup vs baseline: 7.4097x; 7.4097x over previous
import functools

import jax
import jax.numpy as jnp
from jax import lax
from jax.experimental import pallas as pl
from jax.experimental.pallas import tpu as pltpu
from jax.experimental.pallas import tpu_sc as plsc

D = 64
DH = 128
DO = 64
NC, NS = 2, 16
NW = NC * NS
LANES = 16


def _pack_pair(y):
    lo = lax.bitcast_convert_type(y[:, :D].astype(jnp.bfloat16), jnp.uint16)
    hi = lax.bitcast_convert_type(y[:, D:].astype(jnp.bfloat16), jnp.uint16)
    return lo.astype(jnp.int32) | (hi.astype(jnp.int32) << 16)


def _xform_all_body(p_ref, e_ref, wa_ref, wb_ref, wc_ref, t_ref):
    i = pl.program_id(0)

    @pl.when(i == 0)
    def _():
        t_ref[...] = _pack_pair(
            jnp.dot(p_ref[...], wa_ref[...], preferred_element_type=jnp.float32))

    @pl.when((i >= 1) & (i <= 50))
    def _():
        t_ref[...] = _pack_pair(
            jnp.dot(e_ref[...], wb_ref[...], preferred_element_type=jnp.float32))

    @pl.when(i > 50)
    def _():
        t_ref[...] = _pack_pair(
            jnp.dot(e_ref[...], wc_ref[...], preferred_element_type=jnp.float32))


def _xform_all(pred_padded, entity_table, w1a, w1b, w1c):
    blk = 2000
    n_ent = entity_table.shape[0]
    assert pred_padded.shape[0] == blk and n_ent % blk == 0
    grid = 1 + 2 * (n_ent // blk)
    return pl.pallas_call(
        _xform_all_body,
        grid=(grid,),
        in_specs=[
            pl.BlockSpec((blk, D), lambda i: (0, 0)),
            pl.BlockSpec((blk, D), lambda i: ((i - 1) % 50, 0)),
            pl.BlockSpec((D, DH), lambda i: (0, 0)),
            pl.BlockSpec((D, DH), lambda i: (0, 0)),
            pl.BlockSpec((D, DH), lambda i: (0, 0)),
        ],
        out_specs=pl.BlockSpec((blk, D), lambda i: (i, 0)),
        out_shape=jax.ShapeDtypeStruct((grid * blk, D), jnp.int32),
    )(pred_padded, entity_table, w1a, w1b, w1c)


def _make_gather_sum(tok, chunk, sl, ns):
    rows = tok // 2 // ns
    tpw = rows // (NW // 2)
    nchunk = tpw // chunk
    assert tpw % chunk == 0 and nchunk % 2 == 0 and chunk % 8 == 0
    npair = nchunk // 2
    mesh = plsc.VectorSubcoreMesh(core_axis_name="c", subcore_axis_name="s")

    buf_set = [
        pltpu.VMEM((chunk,), jnp.int32),
        pltpu.VMEM((chunk,), jnp.int32),
        pltpu.VMEM((chunk,), jnp.int32),
        pltpu.VMEM((chunk, D), jnp.int32),
        pltpu.VMEM((chunk, D), jnp.int32),
        pltpu.VMEM((chunk, D), jnp.int32),
        pltpu.VMEM((chunk, D), jnp.int32),
        pltpu.SemaphoreType.DMA,
        pltpu.SemaphoreType.DMA,
        pltpu.SemaphoreType.DMA,
    ]

    @functools.partial(
        pl.kernel,
        out_type=jax.ShapeDtypeStruct((rows, DH), jnp.int32),
        mesh=mesh,
        scratch_types=buf_set + buf_set,
        compiler_params=pltpu.CompilerParams(
            needs_layout_passes=False, use_tc_tiling_on_sc=False),
    )
    def gather_sum(pidx_hbm, a1_hbm, a2_hbm, tall_hbm, out_hbm, *scratch):
        wid = lax.axis_index("s") * NC + lax.axis_index("c")
        half = wid // (NW // 2)
        u = wid % (NW // 2)
        base = half * (tok // 2) + sl * rows + u * tpw
        rbase = u * tpw
        coff = half * D
        sets = (scratch[:10], scratch[10:])

        def idx_start(s, c):
            ip, i1, i2, sem_i = s[0], s[1], s[2], s[7]
            off = base + c * chunk
            pltpu.async_copy(pidx_hbm.at[pl.ds(off, chunk)], ip, sem_i)
            pltpu.async_copy(a1_hbm.at[pl.ds(off, chunk)], i1, sem_i)
            pltpu.async_copy(a2_hbm.at[pl.ds(off, chunk)], i2, sem_i)

        def gather_start(s):
            ip, i1, i2, g0, g1, g2, _, sem_i, sem_g, _ = s
            pltpu.make_async_copy(pidx_hbm.at[pl.ds(0, chunk)], ip, sem_i).wait()
            pltpu.make_async_copy(a1_hbm.at[pl.ds(0, chunk)], i1, sem_i).wait()
            pltpu.make_async_copy(a2_hbm.at[pl.ds(0, chunk)], i2, sem_i).wait()
            pltpu.async_copy(tall_hbm.at[ip], g0, sem_g)
            pltpu.async_copy(tall_hbm.at[i1], g1, sem_g)
            pltpu.async_copy(tall_hbm.at[i2], g2, sem_g)

        def wait_gathers(s):
            ip, i1, i2, g0, g1, g2, _, _, sem_g, _ = s
            pltpu.make_async_copy(tall_hbm.at[ip], g0, sem_g).wait()
            pltpu.make_async_copy(tall_hbm.at[i1], g1, sem_g).wait()
            pltpu.make_async_copy(tall_hbm.at[i2], g2, sem_g).wait()

        def wait_scatter(s):
            ob, sem_s = s[6], s[9]
            pltpu.make_async_copy(
                ob, out_hbm.at[pl.ds(0, chunk), pl.ds(coff, D)], sem_s).wait()

        def process(s, c, k):
            g0, g1, g2, ob, sem_s = s[3], s[4], s[5], s[6], s[9]

            @pl.when(k > 0)
            def _():
                wait_scatter(s)

            wait_gathers(s)

            @pl.when(k < npair - 1)
            def _():
                idx_start(s, c + 2)

            @plsc.parallel_loop(0, chunk, unroll=4)
            def _(i):
                for j in range(D // LANES):
                    sl = pl.ds(j * LANES, LANES)
                    a = plsc.bitcast(g0[i, sl], jnp.bfloat16)
                    b = plsc.bitcast(g1[i, sl], jnp.bfloat16)
                    cc = plsc.bitcast(g2[i, sl], jnp.bfloat16)
                    ob[i, sl] = plsc.bitcast(a + b + cc, jnp.int32)

            pltpu.async_copy(
                ob, out_hbm.at[pl.ds(rbase + c * chunk, chunk), pl.ds(coff, D)],
                sem_s)

            @pl.when(k < npair - 1)
            def _():
                gather_start(s)

        idx_start(sets[0], 0)
        idx_start(sets[1], 1)
        gather_start(sets[0])
        gather_start(sets[1])

        def pair_body(k, carry):
            process(sets[0], 2 * k, k)
            process(sets[1], 2 * k + 1, k)
            return carry

        lax.fori_loop(0, npair, pair_body, 0)
        wait_scatter(sets[0])
        wait_scatter(sets[1])

    return gather_sum


def _mlp_body(h_ref, b1_ref, w2_ref, b2_ref, o_ref):
    j = pl.program_id(1)
    x = h_ref[...]
    lo = lax.bitcast_convert_type(x << 16, jnp.float32)
    hi = lax.bitcast_convert_type(x & jnp.int32(-65536), jnp.float32)
    xl = jnp.where(j == 0, lo[:, :D], lo[:, D:])
    xh = jnp.where(j == 0, hi[:, :D], hi[:, D:])
    h = jnp.concatenate([xl, xh], axis=1) + b1_ref[...]
    g = 0.5 * h * (1.0 + lax.erf(h * 0.7071067811865476))
    o_ref[...] = jnp.dot(g, w2_ref[...],
                         preferred_element_type=jnp.float32) + b2_ref[...]


def _mlp_tail(h2, b1, w2, b2, tok, sl, ns, prev=None):
    rows = h2.shape[0]
    blk = 2048
    assert rows % blk == 0
    grid = rows // blk
    gtot = tok // 2 // blk
    in_specs = [
        pl.BlockSpec((blk, DH), lambda i, j: (i, 0)),
        pl.BlockSpec((1, DH), lambda i, j: (0, 0)),
        pl.BlockSpec((DH, DO), lambda i, j: (0, 0)),
        pl.BlockSpec((1, DO), lambda i, j: (0, 0)),
    ]
    args = [h2, b1, w2, b2]
    kw = {}
    if prev is not None:
        in_specs.append(pl.BlockSpec(memory_space=pl.ANY))
        args.append(prev)
        kw["input_output_aliases"] = {4: 0}
    base_blk = sl * grid

    def body(h_ref, b1_ref, w2_ref, b2_ref, *rest):
        _mlp_body(h_ref, b1_ref, w2_ref, b2_ref, rest[-1])

    return pl.pallas_call(
        body,
        grid=(grid, 2),
        in_specs=in_specs,
        out_specs=pl.BlockSpec((blk, DO), lambda i, j: (j * gtot + base_blk + i, 0)),
        out_shape=jax.ShapeDtypeStruct((tok, DO), jnp.float32),
        **kw,
    )(*args)


def kernel(predicate_idx, arg1_idx, arg2_idx, entity_table, predicate_table,
           W1, b1, W2, b2):
    B, L = predicate_idx.shape
    tok = B * L

    pred_padded = jnp.pad(predicate_table, ((0, 1000), (0, 0)))
    tall = _xform_all(pred_padded, entity_table,
                      W1[0:D], W1[D:2 * D], W1[2 * D:3 * D])

    pidx = predicate_idx.reshape(tok).astype(jnp.int32)
    a1 = arg1_idx.reshape(tok).astype(jnp.int32) + 2000
    a2 = arg2_idx.reshape(tok).astype(jnp.int32) + 102000

    b1r, b2r = b1.reshape(1, DH), b2.reshape(1, DO)
    ns = 2
    h0 = _make_gather_sum(tok, 128, 0, ns)(pidx, a1, a2, tall)
    h1 = _make_gather_sum(tok, 128, 1, ns)(pidx, a1, a2, tall)
    o0 = _mlp_tail(h0, b1r, W2, b2r, tok, 0, ns)
    out = _mlp_tail(h1, b1r, W2, b2r, tok, 1, ns, prev=o0)
    return out.reshape(B, L, DO)


# speedup vs baseline: 7.7727x; 1.0490x over previous
import functools

import jax
import jax.numpy as jnp
from jax import lax
from jax.experimental import pallas as pl
from jax.experimental.pallas import tpu as pltpu
from jax.experimental.pallas import tpu_sc as plsc

D = 64
DH = 128
DO = 64
NC, NS = 2, 16
NW = NC * NS
LANES = 16


def _pack_pair(y):
    lo = lax.bitcast_convert_type(y[:, :D].astype(jnp.bfloat16), jnp.uint16)
    hi = lax.bitcast_convert_type(y[:, D:].astype(jnp.bfloat16), jnp.uint16)
    return lo.astype(jnp.int32) | (hi.astype(jnp.int32) << 16)


def _xform_all_body(p_ref, e_ref, wa_ref, wb_ref, wc_ref, t_ref):
    i = pl.program_id(0)

    @pl.when(i == 0)
    def _():
        t_ref[...] = _pack_pair(
            jnp.dot(p_ref[...], wa_ref[...], preferred_element_type=jnp.float32))

    @pl.when((i >= 1) & (i <= 50))
    def _():
        t_ref[...] = _pack_pair(
            jnp.dot(e_ref[...], wb_ref[...], preferred_element_type=jnp.float32))

    @pl.when(i > 50)
    def _():
        t_ref[...] = _pack_pair(
            jnp.dot(e_ref[...], wc_ref[...], preferred_element_type=jnp.float32))


def _xform_all(pred_padded, entity_table, w1a, w1b, w1c):
    blk = 2000
    n_ent = entity_table.shape[0]
    assert pred_padded.shape[0] == blk and n_ent % blk == 0
    grid = 1 + 2 * (n_ent // blk)
    return pl.pallas_call(
        _xform_all_body,
        grid=(grid,),
        in_specs=[
            pl.BlockSpec((blk, D), lambda i: (0, 0)),
            pl.BlockSpec((blk, D), lambda i: ((i - 1) % 50, 0)),
            pl.BlockSpec((D, DH), lambda i: (0, 0)),
            pl.BlockSpec((D, DH), lambda i: (0, 0)),
            pl.BlockSpec((D, DH), lambda i: (0, 0)),
        ],
        out_specs=pl.BlockSpec((blk, D), lambda i: (i, 0)),
        out_shape=jax.ShapeDtypeStruct((grid * blk, D), jnp.int32),
    )(pred_padded, entity_table, w1a, w1b, w1c)


def _make_gather_sum(tok, chunk, sl, ns):
    rows = tok // 2 // ns
    tpw = rows // (NW // 2)
    nchunk = tpw // chunk
    assert tpw % chunk == 0 and nchunk % 2 == 0 and chunk % 8 == 0
    npair = nchunk // 2
    mesh = plsc.VectorSubcoreMesh(core_axis_name="c", subcore_axis_name="s")

    buf_set = [
        pltpu.VMEM((chunk,), jnp.int32),
        pltpu.VMEM((chunk,), jnp.int32),
        pltpu.VMEM((chunk,), jnp.int32),
        pltpu.VMEM((chunk, D), jnp.int32),
        pltpu.VMEM((chunk, D), jnp.int32),
        pltpu.VMEM((chunk, D), jnp.int32),
        pltpu.VMEM((chunk, D), jnp.int32),
        pltpu.SemaphoreType.DMA,
        pltpu.SemaphoreType.DMA,
        pltpu.SemaphoreType.DMA,
    ]

    @functools.partial(
        pl.kernel,
        out_type=jax.ShapeDtypeStruct((rows, DH), jnp.int32),
        mesh=mesh,
        scratch_types=buf_set + buf_set,
        compiler_params=pltpu.CompilerParams(
            needs_layout_passes=False, use_tc_tiling_on_sc=False),
    )
    def gather_sum(pidx_hbm, a1_hbm, a2_hbm, tall_hbm, out_hbm, *scratch):
        wid = lax.axis_index("s") * NC + lax.axis_index("c")
        half = wid // (NW // 2)
        u = wid % (NW // 2)
        base = half * (tok // 2) + sl * rows + u * tpw
        rbase = u * tpw
        coff = half * D
        sets = (scratch[:10], scratch[10:])

        def idx_start(s, c):
            ip, i1, i2, sem_i = s[0], s[1], s[2], s[7]
            off = base + c * chunk
            pltpu.async_copy(pidx_hbm.at[pl.ds(off, chunk)], ip, sem_i)
            pltpu.async_copy(a1_hbm.at[pl.ds(off, chunk)], i1, sem_i)
            pltpu.async_copy(a2_hbm.at[pl.ds(off, chunk)], i2, sem_i)

        def gather_start(s):
            ip, i1, i2, g0, g1, g2, _, sem_i, sem_g, _ = s
            pltpu.make_async_copy(pidx_hbm.at[pl.ds(0, chunk)], ip, sem_i).wait()
            pltpu.make_async_copy(a1_hbm.at[pl.ds(0, chunk)], i1, sem_i).wait()
            pltpu.make_async_copy(a2_hbm.at[pl.ds(0, chunk)], i2, sem_i).wait()
            pltpu.async_copy(tall_hbm.at[ip], g0, sem_g)
            pltpu.async_copy(tall_hbm.at[i1], g1, sem_g)
            pltpu.async_copy(tall_hbm.at[i2], g2, sem_g)

        def wait_gathers(s):
            ip, i1, i2, g0, g1, g2, _, _, sem_g, _ = s
            pltpu.make_async_copy(tall_hbm.at[ip], g0, sem_g).wait()
            pltpu.make_async_copy(tall_hbm.at[i1], g1, sem_g).wait()
            pltpu.make_async_copy(tall_hbm.at[i2], g2, sem_g).wait()

        def wait_scatter(s):
            ob, sem_s = s[6], s[9]
            pltpu.make_async_copy(
                ob, out_hbm.at[pl.ds(0, chunk), pl.ds(coff, D)], sem_s).wait()

        def process(s, c, k):
            g0, g1, g2, ob, sem_s = s[3], s[4], s[5], s[6], s[9]

            @pl.when(k > 0)
            def _():
                wait_scatter(s)

            wait_gathers(s)

            @pl.when(k < npair - 1)
            def _():
                idx_start(s, c + 2)

            @plsc.parallel_loop(0, chunk, unroll=4)
            def _(i):
                for j in range(D // LANES):
                    sl = pl.ds(j * LANES, LANES)
                    a = plsc.bitcast(g0[i, sl], jnp.bfloat16)
                    b = plsc.bitcast(g1[i, sl], jnp.bfloat16)
                    cc = plsc.bitcast(g2[i, sl], jnp.bfloat16)
                    ob[i, sl] = plsc.bitcast(a + b + cc, jnp.int32)

            pltpu.async_copy(
                ob, out_hbm.at[pl.ds(rbase + c * chunk, chunk), pl.ds(coff, D)],
                sem_s)

            @pl.when(k < npair - 1)
            def _():
                gather_start(s)

        idx_start(sets[0], 0)
        idx_start(sets[1], 1)
        gather_start(sets[0])
        gather_start(sets[1])

        def pair_body(k, carry):
            process(sets[0], 2 * k, k)
            process(sets[1], 2 * k + 1, k)
            return carry

        lax.fori_loop(0, npair, pair_body, 0)
        wait_scatter(sets[0])
        wait_scatter(sets[1])

    return gather_sum


def _mlp_body(h_ref, b1_ref, w2_ref, b2_ref, o_ref):
    j = pl.program_id(1)
    x = h_ref[...]
    lo = lax.bitcast_convert_type(x << 16, jnp.float32)
    hi = lax.bitcast_convert_type(x & jnp.int32(-65536), jnp.float32)
    xl = jnp.where(j == 0, lo[:, :D], lo[:, D:])
    xh = jnp.where(j == 0, hi[:, :D], hi[:, D:])
    h = jnp.concatenate([xl, xh], axis=1) + b1_ref[...]
    g = 0.5 * h * (1.0 + lax.erf(h * 0.7071067811865476))
    o_ref[...] = jnp.dot(g, w2_ref[...],
                         preferred_element_type=jnp.float32) + b2_ref[...]


def _mlp_tail(h2, b1, w2, b2, tok, sl, ns, prev=None):
    rows = h2.shape[0]
    blk = 2048
    assert rows % blk == 0
    grid = rows // blk
    gtot = tok // 2 // blk
    in_specs = [
        pl.BlockSpec((blk, DH), lambda i, j: (i, 0)),
        pl.BlockSpec((1, DH), lambda i, j: (0, 0)),
        pl.BlockSpec((DH, DO), lambda i, j: (0, 0)),
        pl.BlockSpec((1, DO), lambda i, j: (0, 0)),
    ]
    args = [h2, b1, w2, b2]
    kw = {}
    if prev is not None:
        in_specs.append(pl.BlockSpec(memory_space=pl.ANY))
        args.append(prev)
        kw["input_output_aliases"] = {4: 0}
    base_blk = sl * grid

    def body(h_ref, b1_ref, w2_ref, b2_ref, *rest):
        _mlp_body(h_ref, b1_ref, w2_ref, b2_ref, rest[-1])

    return pl.pallas_call(
        body,
        grid=(grid, 2),
        in_specs=in_specs,
        out_specs=pl.BlockSpec((blk, DO), lambda i, j: (j * gtot + base_blk + i, 0)),
        out_shape=jax.ShapeDtypeStruct((tok, DO), jnp.float32),
        **kw,
    )(*args)


def kernel(predicate_idx, arg1_idx, arg2_idx, entity_table, predicate_table,
           W1, b1, W2, b2):
    B, L = predicate_idx.shape
    tok = B * L

    pred_padded = jnp.pad(predicate_table, ((0, 1000), (0, 0)))
    tall = _xform_all(pred_padded, entity_table,
                      W1[0:D], W1[D:2 * D], W1[2 * D:3 * D])

    pidx = predicate_idx.reshape(tok).astype(jnp.int32)
    a1 = arg1_idx.reshape(tok).astype(jnp.int32) + 2000
    a2 = arg2_idx.reshape(tok).astype(jnp.int32) + 102000

    b1r, b2r = b1.reshape(1, DH), b2.reshape(1, DO)
    ns = 4
    hs = [_make_gather_sum(tok, 128, sl, ns)(pidx, a1, a2, tall)
          for sl in range(ns)]
    out = _mlp_tail(hs[0], b1r, W2, b2r, tok, 0, ns)
    for sl in range(1, ns):
        out = _mlp_tail(hs[sl], b1r, W2, b2r, tok, sl, ns, prev=out)
    return out.reshape(B, L, DO)


# speedup vs baseline: 8.2630x; 1.0631x over previous
import functools

import jax
import jax.numpy as jnp
from jax import lax
from jax.experimental import pallas as pl
from jax.experimental.pallas import tpu as pltpu
from jax.experimental.pallas import tpu_sc as plsc

D = 64
DH = 128
DO = 64
NC, NS = 2, 16
NW = NC * NS
LANES = 16


def _pack_pair(y):
    lo = lax.bitcast_convert_type(y[:, :D].astype(jnp.bfloat16), jnp.uint16)
    hi = lax.bitcast_convert_type(y[:, D:].astype(jnp.bfloat16), jnp.uint16)
    return lo.astype(jnp.int32) | (hi.astype(jnp.int32) << 16)


def _xform_all_body(p_ref, e_ref, wa_ref, wb_ref, wc_ref, t_ref):
    i = pl.program_id(0)

    @pl.when(i == 0)
    def _():
        t_ref[...] = _pack_pair(
            jnp.dot(p_ref[...], wa_ref[...], preferred_element_type=jnp.float32))

    @pl.when((i >= 1) & (i <= 50))
    def _():
        t_ref[...] = _pack_pair(
            jnp.dot(e_ref[...], wb_ref[...], preferred_element_type=jnp.float32))

    @pl.when(i > 50)
    def _():
        t_ref[...] = _pack_pair(
            jnp.dot(e_ref[...], wc_ref[...], preferred_element_type=jnp.float32))


def _xform_all(pred_padded, entity_table, w1a, w1b, w1c):
    blk = 2000
    n_ent = entity_table.shape[0]
    assert pred_padded.shape[0] == blk and n_ent % blk == 0
    grid = 1 + 2 * (n_ent // blk)
    return pl.pallas_call(
        _xform_all_body,
        grid=(grid,),
        in_specs=[
            pl.BlockSpec((blk, D), lambda i: (0, 0)),
            pl.BlockSpec((blk, D), lambda i: ((i - 1) % 50, 0)),
            pl.BlockSpec((D, DH), lambda i: (0, 0)),
            pl.BlockSpec((D, DH), lambda i: (0, 0)),
            pl.BlockSpec((D, DH), lambda i: (0, 0)),
        ],
        out_specs=pl.BlockSpec((blk, D), lambda i: (i, 0)),
        out_shape=jax.ShapeDtypeStruct((grid * blk, D), jnp.int32),
    )(pred_padded, entity_table, w1a, w1b, w1c)


def _make_gather_sum(tok, chunk, sl, ns):
    rows = tok // 2 // ns
    tpw = rows // (NW // 2)
    nchunk = tpw // chunk
    assert tpw % chunk == 0 and nchunk % 2 == 0 and chunk % 8 == 0
    npair = nchunk // 2
    mesh = plsc.VectorSubcoreMesh(core_axis_name="c", subcore_axis_name="s")

    buf_set = [
        pltpu.VMEM((chunk,), jnp.int32),
        pltpu.VMEM((chunk,), jnp.int32),
        pltpu.VMEM((chunk,), jnp.int32),
        pltpu.VMEM((chunk, D), jnp.int32),
        pltpu.VMEM((chunk, D), jnp.int32),
        pltpu.VMEM((chunk, D), jnp.int32),
        pltpu.VMEM((chunk, D), jnp.int32),
        pltpu.SemaphoreType.DMA,
        pltpu.SemaphoreType.DMA,
        pltpu.SemaphoreType.DMA,
    ]

    @functools.partial(
        pl.kernel,
        out_type=jax.ShapeDtypeStruct((rows, DH), jnp.int32),
        mesh=mesh,
        scratch_types=buf_set + buf_set,
        compiler_params=pltpu.CompilerParams(
            needs_layout_passes=False, use_tc_tiling_on_sc=False),
    )
    def gather_sum(pidx_hbm, a1_hbm, a2_hbm, tall_hbm, out_hbm, *scratch):
        wid = lax.axis_index("s") * NC + lax.axis_index("c")
        half = wid // (NW // 2)
        u = wid % (NW // 2)
        base = half * (tok // 2) + sl * rows + u * tpw
        rbase = u * tpw
        coff = half * D
        sets = (scratch[:10], scratch[10:])

        def idx_start(s, c):
            ip, i1, i2, sem_i = s[0], s[1], s[2], s[7]
            off = base + c * chunk
            pltpu.async_copy(pidx_hbm.at[pl.ds(off, chunk)], ip, sem_i)
            pltpu.async_copy(a1_hbm.at[pl.ds(off, chunk)], i1, sem_i)
            pltpu.async_copy(a2_hbm.at[pl.ds(off, chunk)], i2, sem_i)

        def gather_start(s):
            ip, i1, i2, g0, g1, g2, _, sem_i, sem_g, _ = s
            pltpu.make_async_copy(pidx_hbm.at[pl.ds(0, chunk)], ip, sem_i).wait()
            pltpu.make_async_copy(a1_hbm.at[pl.ds(0, chunk)], i1, sem_i).wait()
            pltpu.make_async_copy(a2_hbm.at[pl.ds(0, chunk)], i2, sem_i).wait()
            pltpu.async_copy(tall_hbm.at[ip], g0, sem_g)
            pltpu.async_copy(tall_hbm.at[i1], g1, sem_g)
            pltpu.async_copy(tall_hbm.at[i2], g2, sem_g)

        def wait_gathers(s):
            ip, i1, i2, g0, g1, g2, _, _, sem_g, _ = s
            pltpu.make_async_copy(tall_hbm.at[ip], g0, sem_g).wait()
            pltpu.make_async_copy(tall_hbm.at[i1], g1, sem_g).wait()
            pltpu.make_async_copy(tall_hbm.at[i2], g2, sem_g).wait()

        def wait_scatter(s):
            ob, sem_s = s[6], s[9]
            pltpu.make_async_copy(
                ob, out_hbm.at[pl.ds(0, chunk), pl.ds(coff, D)], sem_s).wait()

        def process(s, c, k):
            g0, g1, g2, ob, sem_s = s[3], s[4], s[5], s[6], s[9]

            @pl.when(k > 0)
            def _():
                wait_scatter(s)

            wait_gathers(s)

            @pl.when(k < npair - 1)
            def _():
                idx_start(s, c + 2)

            @plsc.parallel_loop(0, chunk, unroll=4)
            def _(i):
                for j in range(D // LANES):
                    sl = pl.ds(j * LANES, LANES)
                    a = plsc.bitcast(g0[i, sl], jnp.bfloat16)
                    b = plsc.bitcast(g1[i, sl], jnp.bfloat16)
                    cc = plsc.bitcast(g2[i, sl], jnp.bfloat16)
                    ob[i, sl] = plsc.bitcast(a + b + cc, jnp.int32)

            pltpu.async_copy(
                ob, out_hbm.at[pl.ds(rbase + c * chunk, chunk), pl.ds(coff, D)],
                sem_s)

            @pl.when(k < npair - 1)
            def _():
                gather_start(s)

        idx_start(sets[0], 0)
        idx_start(sets[1], 1)
        gather_start(sets[0])
        gather_start(sets[1])

        def pair_body(k, carry):
            process(sets[0], 2 * k, k)
            process(sets[1], 2 * k + 1, k)
            return carry

        lax.fori_loop(0, npair, pair_body, 0)
        wait_scatter(sets[0])
        wait_scatter(sets[1])

    return gather_sum


def _mlp_body(h_ref, b1_ref, w2_ref, b2_ref, o_ref):
    j = pl.program_id(1)
    x = h_ref[...]
    lo = lax.bitcast_convert_type(x << 16, jnp.float32)
    hi = lax.bitcast_convert_type(x & jnp.int32(-65536), jnp.float32)
    xl = jnp.where(j == 0, lo[:, :D], lo[:, D:])
    xh = jnp.where(j == 0, hi[:, :D], hi[:, D:])
    h = jnp.concatenate([xl, xh], axis=1) + b1_ref[...]
    g = 0.5 * h * (1.0 + lax.erf(h * 0.7071067811865476))
    o_ref[...] = jnp.dot(g, w2_ref[...],
                         preferred_element_type=jnp.float32) + b2_ref[...]


def _mlp_tail(h2, b1, w2, b2, tok, sl, ns, prev=None):
    rows = h2.shape[0]
    blk = 2048
    assert rows % blk == 0
    grid = rows // blk
    gtot = tok // 2 // blk
    in_specs = [
        pl.BlockSpec((blk, DH), lambda i, j: (i, 0)),
        pl.BlockSpec((1, DH), lambda i, j: (0, 0)),
        pl.BlockSpec((DH, DO), lambda i, j: (0, 0)),
        pl.BlockSpec((1, DO), lambda i, j: (0, 0)),
    ]
    args = [h2, b1, w2, b2]
    kw = {}
    if prev is not None:
        in_specs.append(pl.BlockSpec(memory_space=pl.ANY))
        args.append(prev)
        kw["input_output_aliases"] = {4: 0}
    base_blk = sl * grid

    def body(h_ref, b1_ref, w2_ref, b2_ref, *rest):
        _mlp_body(h_ref, b1_ref, w2_ref, b2_ref, rest[-1])

    return pl.pallas_call(
        body,
        grid=(grid, 2),
        in_specs=in_specs,
        out_specs=pl.BlockSpec((blk, DO), lambda i, j: (j * gtot + base_blk + i, 0)),
        out_shape=jax.ShapeDtypeStruct((tok, DO), jnp.float32),
        **kw,
    )(*args)


def kernel(predicate_idx, arg1_idx, arg2_idx, entity_table, predicate_table,
           W1, b1, W2, b2):
    B, L = predicate_idx.shape
    tok = B * L

    pred_padded = jnp.pad(predicate_table, ((0, 1000), (0, 0)))
    tall = _xform_all(pred_padded, entity_table,
                      W1[0:D], W1[D:2 * D], W1[2 * D:3 * D])

    pidx = predicate_idx.reshape(tok).astype(jnp.int32)
    a1 = arg1_idx.reshape(tok).astype(jnp.int32) + 2000
    a2 = arg2_idx.reshape(tok).astype(jnp.int32) + 102000

    b1r, b2r = b1.reshape(1, DH), b2.reshape(1, DO)
    ns = 8
    hs = [_make_gather_sum(tok, 64, sl, ns)(pidx, a1, a2, tall)
          for sl in range(ns)]
    out = _mlp_tail(hs[0], b1r, W2, b2r, tok, 0, ns)
    for sl in range(1, ns):
        out = _mlp_tail(hs[sl], b1r, W2, b2r, tok, sl, ns, prev=out)
    return out.reshape(B, L, DO)
